# baseline (speedup 1.0000x reference)
import jax
import jax.numpy as jnp
from jax.experimental import pallas as pl
from jax.experimental.pallas import tpu as pltpu

MAX_H_BOUND = 1080.0
MAX_W_BOUND = 1920.0

TB = 16

_DN = (((0,), (0,)), ((), ()))


def _bf16x3(x):
    hi = x.astype(jnp.bfloat16)
    r = x - hi.astype(jnp.float32)
    mid = r.astype(jnp.bfloat16)
    lo = (r - mid.astype(jnp.float32)).astype(jnp.bfloat16)
    return hi, mid, lo


def _lane_min_dense(x):
    n = x.shape[1]
    while n > 128:
        n //= 2
        x = jnp.minimum(x[:, :n], x[:, n:])
    return _sublane_min(jnp.transpose(x))


def _sublane_min(x):
    n = x.shape[0]
    while n > 1:
        n //= 2
        x = jnp.minimum(x[:n, :], x[n:, :])
    return x


def _chamfer_steps_kernel(p_ref, v_ref, q_ref, o_ref):
    C = p_ref.shape[3]
    racc = jnp.zeros((1, C), jnp.float32)
    cacc = jnp.zeros((1, C), jnp.float32)
    for k in range(TB):
        pc = p_ref[0, k]
        vv = v_ref[0, k]
        qc = q_ref[0, k]

        s = pc + vv
        sx = jnp.clip(s[0:1, :], 0.0, MAX_H_BOUND)
        sy = jnp.clip(s[1:2, :], 0.0, MAX_W_BOUND)
        qx = qc[0:1, :]
        qy = qc[1:2, :]

        p2 = sx * sx + sy * sy
        q2 = qx * qx + qy * qy

        ones = jnp.ones((1, C), jnp.bfloat16)
        p2h, p2m, p2l = _bf16x3(p2)
        q2h, q2m, q2l = _bf16x3(q2)

        lhs = jnp.concatenate(
            [
                (-2.0 * sx).astype(jnp.bfloat16),
                (-2.0 * sy).astype(jnp.bfloat16),
                p2h, p2m, p2l,
                ones, ones, ones,
            ],
            axis=0,
        )
        rhs = jnp.concatenate(
            [
                qc.astype(jnp.bfloat16),
                ones, ones, ones,
                q2h, q2m, q2l,
            ],
            axis=0,
        )
        d2 = jax.lax.dot_general(lhs, rhs, _DN, preferred_element_type=jnp.float32)

        rmin = _lane_min_dense(d2)
        cmin = _sublane_min(d2)

        racc = racc + jnp.sqrt(jnp.maximum(rmin, 0.0))
        cacc = cacc + jnp.sqrt(jnp.maximum(cmin, 0.0))

    partial = jnp.sum(racc) + jnp.sum(cacc)
    o_ref[...] = jnp.full(o_ref.shape, partial, dtype=jnp.float32)


def _chamfer_total(center_tensor, velocity_vector):
    N, T, C, _ = center_tensor.shape
    Tm = T - 1
    ct = center_tensor.transpose(0, 1, 3, 2)
    vt = velocity_vector.transpose(0, 1, 3, 2)

    out = pl.pallas_call(
        _chamfer_steps_kernel,
        grid=(N, Tm // TB),
        in_specs=[
            pl.BlockSpec((1, TB, 2, C), lambda n, t: (n, t, 0, 0)),
            pl.BlockSpec((1, TB, 2, C), lambda n, t: (n, t, 0, 0)),
            pl.BlockSpec((1, TB, 2, C), lambda n, t: (n, t, 0, 0)),
        ],
        out_specs=pl.BlockSpec((1, 1, 8, 128), lambda n, t: (n, t, 0, 0)),
        out_shape=jax.ShapeDtypeStruct((N, Tm // TB, 8, 128), jnp.float32),
        compiler_params=pltpu.CompilerParams(
            dimension_semantics=("parallel", "arbitrary"),
        ),
        name="chamfer_steps",
    )(ct[:, :-1], vt[:, :-1], ct[:, 1:])

    return jnp.sum(out[:, :, 0, 0])


def kernel(center_tensor, velocity_vector):
    N, T, C, _ = center_tensor.shape
    Tm = T - 1
    scale = 0.5 / (N * C * Tm)

    devices = jax.devices()[:2]
    if len(devices) == 2 and N % 2 == 0:
        mesh = jax.sharding.Mesh(devices, ("d",))
        p = jax.sharding.PartitionSpec

        def _shard_fn(c, v):
            return jax.lax.psum(_chamfer_total(c, v), "d")

        total = jax.shard_map(
            _shard_fn,
            mesh=mesh,
            in_specs=(p("d"), p("d")),
            out_specs=p(),
            check_vma=False,
        )(center_tensor, velocity_vector)
    else:
        total = _chamfer_total(center_tensor, velocity_vector)

    return total * scale


# speedup vs baseline: 5.6249x; 5.6249x over previous
import jax
import jax.numpy as jnp
from jax.experimental import pallas as pl
from jax.experimental.pallas import tpu as pltpu

MAX_H_BOUND = 1080.0
MAX_W_BOUND = 1920.0

TB = 32

_DN = (((0,), (0,)), ((), ()))


def _bf16x3(x):
    hi = x.astype(jnp.bfloat16)
    r = x - hi.astype(jnp.float32)
    mid = r.astype(jnp.bfloat16)
    lo = (r - mid.astype(jnp.float32)).astype(jnp.bfloat16)
    return hi, mid, lo


def _lane_min_dense(x):
    n = x.shape[1]
    while n > 128:
        n //= 2
        x = jnp.minimum(x[:, :n], x[:, n:])
    return _sublane_min(jnp.transpose(x))


def _sublane_min(x):
    n = x.shape[0]
    while n > 1:
        n //= 2
        x = jnp.minimum(x[:n, :], x[n:, :])
    return x


def _chamfer_steps_kernel(p_ref, v_ref, q_ref, o_ref):
    C = p_ref.shape[3]
    racc = jnp.zeros((1, C), jnp.float32)
    cacc = jnp.zeros((1, C), jnp.float32)
    for k in range(TB):
        pc = p_ref[0, k]
        vv = v_ref[0, k]
        qc = q_ref[0, k]

        s = pc + vv
        sx = jnp.clip(s[0:1, :], 0.0, MAX_H_BOUND)
        sy = jnp.clip(s[1:2, :], 0.0, MAX_W_BOUND)
        qx = qc[0:1, :]
        qy = qc[1:2, :]

        p2 = sx * sx + sy * sy
        q2 = qx * qx + qy * qy

        ones = jnp.ones((1, C), jnp.bfloat16)
        p2h, p2m, p2l = _bf16x3(p2)
        q2h, q2m, q2l = _bf16x3(q2)

        lhs = jnp.concatenate(
            [
                (-2.0 * sx).astype(jnp.bfloat16),
                (-2.0 * sy).astype(jnp.bfloat16),
                p2h, p2m, p2l,
                ones, ones, ones,
            ],
            axis=0,
        )
        rhs = jnp.concatenate(
            [
                qc.astype(jnp.bfloat16),
                ones, ones, ones,
                q2h, q2m, q2l,
            ],
            axis=0,
        )
        d2 = jax.lax.dot_general(lhs, rhs, _DN, preferred_element_type=jnp.float32)

        rmin = _lane_min_dense(d2)
        cmin = _sublane_min(d2)

        racc = racc + jnp.sqrt(jnp.maximum(rmin, 0.0))
        cacc = cacc + jnp.sqrt(jnp.maximum(cmin, 0.0))

    partial = jnp.sum(racc) + jnp.sum(cacc)
    o_ref[...] = jnp.full(o_ref.shape, partial, dtype=jnp.float32)


def kernel(center_tensor, velocity_vector):
    N, T, C, _ = center_tensor.shape
    Tm = T - 1
    ct = center_tensor.transpose(0, 1, 3, 2)
    vt = velocity_vector.transpose(0, 1, 3, 2)

    out = pl.pallas_call(
        _chamfer_steps_kernel,
        grid=(N, Tm // TB),
        in_specs=[
            pl.BlockSpec((1, TB, 2, C), lambda n, t: (n, t, 0, 0)),
            pl.BlockSpec((1, TB, 2, C), lambda n, t: (n, t, 0, 0)),
            pl.BlockSpec((1, TB, 2, C), lambda n, t: (n, t, 0, 0)),
        ],
        out_specs=pl.BlockSpec((1, 1, 8, 128), lambda n, t: (n, t, 0, 0)),
        out_shape=jax.ShapeDtypeStruct((N, Tm // TB, 8, 128), jnp.float32),
        compiler_params=pltpu.CompilerParams(
            dimension_semantics=("parallel", "arbitrary"),
        ),
        name="chamfer_steps",
    )(ct[:, :-1], vt[:, :-1], ct[:, 1:])

    total = jnp.sum(out[:, :, 0, 0])
    return total * (0.5 / (N * C * Tm))


# speedup vs baseline: 6.1521x; 1.0937x over previous
import jax
import jax.numpy as jnp
from jax.experimental import pallas as pl
from jax.experimental.pallas import tpu as pltpu

MAX_H_BOUND = 1080.0
MAX_W_BOUND = 1920.0

TB = 32

_DN = (((0,), (0,)), ((), ()))


def _bf16x3(x):
    hi = x.astype(jnp.bfloat16)
    r = x - hi.astype(jnp.float32)
    mid = r.astype(jnp.bfloat16)
    lo = (r - mid.astype(jnp.float32)).astype(jnp.bfloat16)
    return hi, mid, lo


def _lane_min_dense(x):
    n = x.shape[1]
    while n > 128:
        n //= 2
        x = jnp.minimum(x[:, :n], x[:, n:])
    return _sublane_min(jnp.transpose(x))


def _sublane_min(x):
    n = x.shape[0]
    while n > 1:
        n //= 2
        x = jnp.minimum(x[:n, :], x[n:, :])
    return x


def _chamfer_steps_kernel(c_ref, v_ref, o_ref):
    C = c_ref.shape[3]
    racc = jnp.zeros((1, C), jnp.float32)
    cacc = jnp.zeros((1, C), jnp.float32)
    for k in range(TB):
        pc = c_ref[0, k]
        vv = v_ref[0, k]
        qc = c_ref[0, k + 1]

        s = pc + vv
        sx = jnp.clip(s[0:1, :], 0.0, MAX_H_BOUND)
        sy = jnp.clip(s[1:2, :], 0.0, MAX_W_BOUND)
        qx = qc[0:1, :]
        qy = qc[1:2, :]

        p2 = sx * sx + sy * sy
        q2 = qx * qx + qy * qy

        ones = jnp.ones((1, C), jnp.bfloat16)
        p2h, p2m, p2l = _bf16x3(p2)
        q2h, q2m, q2l = _bf16x3(q2)

        lhs = jnp.concatenate(
            [
                (-2.0 * sx).astype(jnp.bfloat16),
                (-2.0 * sy).astype(jnp.bfloat16),
                p2h, p2m, p2l,
                ones, ones, ones,
            ],
            axis=0,
        )
        rhs = jnp.concatenate(
            [
                qc.astype(jnp.bfloat16),
                ones, ones, ones,
                q2h, q2m, q2l,
            ],
            axis=0,
        )
        d2 = jax.lax.dot_general(lhs, rhs, _DN, preferred_element_type=jnp.float32)

        rmin = _lane_min_dense(d2)
        cmin = _sublane_min(d2)

        racc = racc + jnp.sqrt(jnp.maximum(rmin, 0.0))
        cacc = cacc + jnp.sqrt(jnp.maximum(cmin, 0.0))

    partial = jnp.sum(racc) + jnp.sum(cacc)
    o_ref[...] = jnp.full(o_ref.shape, partial, dtype=jnp.float32)


def kernel(center_tensor, velocity_vector):
    N, T, C, _ = center_tensor.shape
    Tm = T - 1
    ct = center_tensor.transpose(0, 1, 3, 2)
    vt = velocity_vector.transpose(0, 1, 3, 2)

    out = pl.pallas_call(
        _chamfer_steps_kernel,
        grid=(N,),
        in_specs=[
            pl.BlockSpec((1, T, 2, C), lambda n: (n, 0, 0, 0)),
            pl.BlockSpec((1, T, 2, C), lambda n: (n, 0, 0, 0)),
        ],
        out_specs=pl.BlockSpec((1, 8, 128), lambda n: (n, 0, 0)),
        out_shape=jax.ShapeDtypeStruct((N, 8, 128), jnp.float32),
        compiler_params=pltpu.CompilerParams(
            dimension_semantics=("parallel",),
        ),
        name="chamfer_steps",
    )(ct, vt)

    total = jnp.sum(out[:, 0, 0])
    return total * (0.5 / (N * C * Tm))
